# Initial kernel scaffold; baseline (speedup 1.0000x reference)
import jax
import jax.numpy as jnp
from jax.experimental import pallas as pl


def kernel(x, embed):
    raise NotImplementedError("write your pallas kernel here")


# speedup vs baseline: 1.4522x; 1.4522x over previous
import functools

import jax
import jax.numpy as jnp
from jax import lax
from jax.experimental import pallas as pl
from jax.experimental.pallas import tpu as pltpu
from jax.experimental.pallas import tpu_sc as plsc

D = 256
K = 8192
N = 16384


NT = 512
KT = 2048


def _argmin_body(x_ref, x2_ref, e_ref, e2_ref, out_ref):
    xb = x_ref[...]
    x2b = x2_ref[...]

    def step(k, carry):
        bv, bi = carry
        off = pl.multiple_of(k * KT, KT)
        eb = e_ref[:, pl.ds(off, KT)]
        s = lax.dot_general(xb.astype(jnp.bfloat16), eb.astype(jnp.bfloat16),
                            (((1,), (0,)), ((), ())),
                            preferred_element_type=jnp.float32)
        d = (x2b - 2.0 * s) + e2_ref[:, pl.ds(off, KT)]
        m = jnp.min(d, axis=1, keepdims=True)
        col = lax.broadcasted_iota(jnp.int32, (NT, KT), 1) + k * KT
        c = jnp.min(jnp.where(d <= m, col, jnp.int32(K)),
                    axis=1, keepdims=True)
        upd = m < bv
        return jnp.where(upd, m, bv), jnp.where(upd, c, bi)

    bv0 = jnp.full((NT, 1), jnp.inf, jnp.float32)
    bi0 = jnp.zeros((NT, 1), jnp.int32)
    _, bi = lax.fori_loop(0, K // KT, step, (bv0, bi0))
    out_ref[...] = bi


def _argmin_call(x_flat, x2, embed, e2):
    return pl.pallas_call(
        _argmin_body,
        grid=(N // NT,),
        in_specs=[
            pl.BlockSpec((NT, D), lambda n: (n, 0)),
            pl.BlockSpec((NT, 1), lambda n: (n, 0)),
            pl.BlockSpec((D, K), lambda n: (0, 0)),
            pl.BlockSpec((1, K), lambda n: (0, 0)),
        ],
        out_specs=pl.BlockSpec((NT, 1), lambda n: (n, 0)),
        out_shape=jax.ShapeDtypeStruct((N, 1), jnp.int32),
        compiler_params=pltpu.CompilerParams(
            dimension_semantics=("arbitrary",),
        ),
    )(x_flat, x2, embed, e2)


NC = 2
NS = 16
NW = NC * NS
BPW = N // NW
CH = 128


def _sc_gather_hist(embed_t, ind, zeros_k, ones_w):
    mesh = plsc.VectorSubcoreMesh(core_axis_name="c", subcore_axis_name="s")

    @functools.partial(
        pl.kernel,
        mesh=mesh,
        out_type=(
            jax.ShapeDtypeStruct((N, D), jnp.float32),
            jax.ShapeDtypeStruct((NC, K), jnp.float32),
        ),
        scratch_types=[
            pltpu.VMEM((BPW,), jnp.int32),
            pltpu.VMEM((CH, D), jnp.float32),
            pltpu.VMEM((BPW,), jnp.float32),
            pltpu.VMEM_SHARED((K,), jnp.float32),
            pltpu.SemaphoreType.DMA,
        ],
    )
    def body(tbl_hbm, idx_hbm, zer_hbm, one_hbm, q_hbm, cnt_hbm,
             idx_v, buf, ones_v, cnts_sh, sem):
        cid = lax.axis_index("c")
        sid = lax.axis_index("s")
        wid = sid * NC + cid
        base = wid * BPW
        pltpu.sync_copy(idx_hbm.at[pl.ds(base, BPW)], idx_v)
        pltpu.sync_copy(one_hbm, ones_v)

        @pl.when(sid == 0)
        def _():
            pltpu.sync_copy(zer_hbm, cnts_sh)

        plsc.subcore_barrier()
        pltpu.sync_copy(ones_v, cnts_sh.at[idx_v], add=True)
        plsc.subcore_barrier()

        @pl.when(sid == 0)
        def _():
            pltpu.sync_copy(cnts_sh, cnt_hbm.at[cid])

        for ch in range(BPW // CH):
            idx_sl = idx_v.at[pl.ds(ch * CH, CH)]
            pltpu.async_copy(tbl_hbm.at[idx_sl], buf, sem).wait()
            pltpu.sync_copy(buf, q_hbm.at[pl.ds(base + ch * CH, CH)])

    return body(embed_t, ind, zeros_k, ones_w)


NB = 16
CT = N // NB


def _fin_body(q_ref, x_ref, cnt_ref, loss_ref, perp_ref):
    b = pl.program_id(0)
    diff = q_ref[...] - x_ref[...]
    part = jnp.sum(diff * diff)

    @pl.when(b == 0)
    def _():
        loss_ref[0, 0] = 0.0

    loss_ref[0, 0] += part

    @pl.when(b == NB - 1)
    def _():
        loss_ref[0, 0] = loss_ref[0, 0] * (1.25 / (N * D))
        p = (cnt_ref[0, :] + cnt_ref[1, :]) * (1.0 / N)
        ent = jnp.sum(p * jnp.log(p + 1e-10))
        perp_ref[0, 0] = jnp.exp(-ent)


def _finalize(q_flat, x_flat, counts):
    return pl.pallas_call(
        _fin_body,
        grid=(NB,),
        in_specs=[
            pl.BlockSpec((CT, D), lambda b: (b, 0)),
            pl.BlockSpec((CT, D), lambda b: (b, 0)),
            pl.BlockSpec((NC, K), lambda b: (0, 0)),
        ],
        out_specs=[
            pl.BlockSpec(memory_space=pltpu.MemorySpace.SMEM),
            pl.BlockSpec(memory_space=pltpu.MemorySpace.SMEM),
        ],
        out_shape=[
            jax.ShapeDtypeStruct((1, 1), jnp.float32),
            jax.ShapeDtypeStruct((1, 1), jnp.float32),
        ],
        compiler_params=pltpu.CompilerParams(
            dimension_semantics=("arbitrary",),
        ),
    )(q_flat, x_flat, counts)


def kernel(x, embed):
    x_p = jnp.transpose(x, (0, 2, 3, 1))
    x_flat = x_p.reshape(-1, D)
    x2 = jnp.sum(x_flat ** 2, axis=1, keepdims=True)
    e2 = jnp.sum(embed ** 2, axis=0, keepdims=True)
    ind = _argmin_call(x_flat, x2, embed, e2).reshape(-1)

    embed_t = embed.T
    zeros_k = jnp.zeros((K,), jnp.float32)
    ones_w = jnp.ones((BPW,), jnp.float32)
    q_flat, counts = _sc_gather_hist(embed_t, ind, zeros_k, ones_w)

    loss11, perp11 = _finalize(q_flat, x_flat, counts)

    quantized_st = jnp.transpose(q_flat.reshape(16, 32, 32, D), (0, 3, 1, 2))
    return (loss11.reshape(()), quantized_st, perp11.reshape(()),
            ind.reshape(16, 32, 32))
